# baseline (speedup 1.0000x reference)
import functools

import jax
import jax.numpy as jnp
from jax import lax
from jax.experimental import pallas as pl
from jax.experimental.pallas import tpu as pltpu
from jax.experimental.pallas import tpu_sc as plsc

NC = 2
NS = 16
L = 16
NW = NC * NS

CH = 64
LN_EPS = 1e-12


def _make_sc_kernel(batch, seq, vocab, d_model):
    assert d_model % L == 0
    nd = d_model // L
    tokens = batch * seq
    assert batch % NW == 0
    bpw = batch // NW
    tpw = tokens // NW
    assert seq % CH == 0
    ncheck = seq // CH

    mesh = plsc.VectorSubcoreMesh(
        core_axis_name="c", subcore_axis_name="s", num_cores=NC, num_subcores=NS
    )

    def body(ids_h, seg_h, tok_h, pos_h, sege_h, gam_h, bet_h, out_h,
             idx_v, segv, rows_v, pos_v, sege_v, gam_v, bet_v, sem):
        w = lax.axis_index("s") * NC + lax.axis_index("c")
        wbase = w * tpw
        pltpu.sync_copy(sege_h, sege_v)
        pltpu.sync_copy(gam_h, gam_v)
        pltpu.sync_copy(bet_h, bet_v)
        iota = lax.iota(jnp.int32, L)
        inv_d = 1.0 / d_model

        def tok_body(t, _):
            segb = plsc.load_gather(segv, [jnp.full((L,), 0, jnp.int32) + t])
            acc = jnp.zeros((L,), jnp.float32)
            acc2 = jnp.zeros((L,), jnp.float32)
            for d in range(nd):
                sl = pl.ds(d * L, L)
                srow = plsc.load_gather(sege_v, [segb, iota + (d * L)])
                v = rows_v[t, sl] + pos_v[t, sl] + srow
                rows_v[t, sl] = v
                acc = acc + v
                acc2 = acc2 + v * v
            totv = jnp.full((L,), jnp.sum(acc), jnp.float32)
            tot2v = jnp.full((L,), jnp.sum(acc2), jnp.float32)
            mv = totv * inv_d
            varv = tot2v * inv_d - mv * mv
            sv = varv + LN_EPS
            iv = plsc.bitcast(sv, jnp.int32)
            iv = jnp.full((L,), 0x5F3759DF, jnp.int32) - jnp.right_shift(iv, 1)
            y = plsc.bitcast(iv, jnp.float32)
            for _ in range(3):
                y = y * (1.5 - 0.5 * sv * y * y)
            bbv = -mv * y
            for d in range(nd):
                sl = pl.ds(d * L, L)
                v = rows_v[t, sl]
                o = v * y + bbv
                rows_v[t, sl] = o * gam_v[sl] + bet_v[sl]
            return 0

        def batch_body(args):
            c, b = args
            g0 = wbase + b * seq + c * CH
            pltpu.sync_copy(ids_h.at[pl.ds(g0, CH)], idx_v)
            pltpu.sync_copy(seg_h.at[pl.ds(g0, CH)], segv)
            pltpu.async_copy(tok_h.at[idx_v], rows_v, sem).wait()
            lax.fori_loop(0, CH, tok_body, 0)
            pltpu.sync_copy(rows_v, out_h.at[pl.ds(g0, CH)])

        def chunk_body(c, _):
            pltpu.sync_copy(pos_h.at[pl.ds(c * CH, CH)], pos_v)

            def bb(b, _):
                batch_body((c, b))
                return 0

            lax.fori_loop(0, bpw, bb, 0)
            return 0

        lax.fori_loop(0, ncheck, chunk_body, 0)

    grid_kernel = pl.kernel(
        body,
        out_type=jax.ShapeDtypeStruct((tokens, d_model), jnp.float32),
        mesh=mesh,
        compiler_params=pltpu.CompilerParams(needs_layout_passes=False),
        scratch_types=[
            pltpu.VMEM((CH,), jnp.int32),
            pltpu.VMEM((CH,), jnp.int32),
            pltpu.VMEM((CH, d_model), jnp.float32),
            pltpu.VMEM((CH, d_model), jnp.float32),
            pltpu.VMEM((2, d_model), jnp.float32),
            pltpu.VMEM((d_model,), jnp.float32),
            pltpu.VMEM((d_model,), jnp.float32),
            pltpu.SemaphoreType.DMA,
        ],
    )
    return grid_kernel


def kernel(input_ids, segment_ids, tok_emb, pos_emb, seg_emb, gamma, beta):
    batch, seq = input_ids.shape
    vocab, d_model = tok_emb.shape
    ids = input_ids.reshape(-1).astype(jnp.int32)
    segs = segment_ids.reshape(-1).astype(jnp.int32)
    sc = _make_sc_kernel(batch, seq, vocab, d_model)
    out = sc(ids, segs, tok_emb, pos_emb, seg_emb, gamma, beta)
    return out.reshape(batch, seq, d_model)


# speedup vs baseline: 4.0869x; 4.0869x over previous
import jax
import jax.numpy as jnp
from jax import lax
from jax.experimental import pallas as pl
from jax.experimental.pallas import tpu as pltpu
from jax.experimental.pallas import tpu_sc as plsc

NC = 2
NS = 16
NW = NC * NS

CHG = 64
RPB = 2
LN_EPS = 1e-12


def _sc_gather(ids, tok_emb):
    tokens, = ids.shape
    vocab, d_model = tok_emb.shape
    assert tokens % NW == 0
    tpw = tokens // NW
    assert tpw % CHG == 0
    nch = tpw // CHG

    mesh = plsc.VectorSubcoreMesh(
        core_axis_name="c", subcore_axis_name="s", num_cores=NC, num_subcores=NS
    )

    def body(ids_h, tok_h, out_h, idx_all, rows_v, gsem, osem):
        w = lax.axis_index("s") * NC + lax.axis_index("c")
        wbase = w * tpw
        pltpu.sync_copy(ids_h.at[pl.ds(wbase, tpw)], idx_all)
        pend_g = None
        pend_o = [None, None]
        for c in range(nch + 1):
            p = c & 1
            if pend_o[p] is not None:
                pend_o[p].wait()
                pend_o[p] = None
            g = None
            if c < nch:
                g = pltpu.async_copy(
                    tok_h.at[idx_all.at[pl.ds(c * CHG, CHG)]], rows_v.at[p], gsem
                )
            if pend_g is not None:
                pend_g.wait()
                pend_o[1 - p] = pltpu.async_copy(
                    rows_v.at[1 - p],
                    out_h.at[pl.ds(wbase + (c - 1) * CHG, CHG)],
                    osem,
                )
            pend_g = g
        for p in range(2):
            if pend_o[p] is not None:
                pend_o[p].wait()

    return pl.kernel(
        body,
        out_type=jax.ShapeDtypeStruct((tokens, d_model), jnp.float32),
        mesh=mesh,
        compiler_params=pltpu.CompilerParams(needs_layout_passes=False),
        scratch_types=[
            pltpu.VMEM((tpw,), jnp.int32),
            pltpu.VMEM((2, CHG, d_model), jnp.float32),
            pltpu.SemaphoreType.DMA,
            pltpu.SemaphoreType.DMA,
        ],
    )(ids, tok_emb)


def _tc_addnorm(gat, segf, pos_emb, seg_emb, gamma, beta):
    batch, seq, d_model = gat.shape

    def body(gat_ref, segf_ref, pos_ref, sege_ref, gam_ref, bet_ref, out_ref):
        base = pos_ref[...] + sege_ref[0, :][None, :]
        diff = (sege_ref[1, :] - sege_ref[0, :])[None, None, :]
        emb = gat_ref[...] + base[None, :, :] + segf_ref[...] * diff
        mean = jnp.mean(emb, axis=-1, keepdims=True)
        cent = emb - mean
        var = jnp.mean(cent * cent, axis=-1, keepdims=True)
        rstd = lax.rsqrt(var + LN_EPS)
        out_ref[...] = (cent * rstd) * gam_ref[0, :][None, None, :] + bet_ref[0, :][None, None, :]

    return pl.pallas_call(
        body,
        grid=(batch // RPB,),
        in_specs=[
            pl.BlockSpec((RPB, seq, d_model), lambda i: (i, 0, 0)),
            pl.BlockSpec((RPB, seq, 1), lambda i: (i, 0, 0)),
            pl.BlockSpec((seq, d_model), lambda i: (0, 0)),
            pl.BlockSpec((2, d_model), lambda i: (0, 0)),
            pl.BlockSpec((1, d_model), lambda i: (0, 0)),
            pl.BlockSpec((1, d_model), lambda i: (0, 0)),
        ],
        out_specs=pl.BlockSpec((RPB, seq, d_model), lambda i: (i, 0, 0)),
        out_shape=jax.ShapeDtypeStruct((batch, seq, d_model), jnp.float32),
    )(gat, segf, pos_emb, seg_emb, gamma.reshape(1, -1), beta.reshape(1, -1))


def kernel(input_ids, segment_ids, tok_emb, pos_emb, seg_emb, gamma, beta):
    batch, seq = input_ids.shape
    _, d_model = tok_emb.shape
    ids = input_ids.reshape(-1).astype(jnp.int32)
    gat = _sc_gather(ids, tok_emb).reshape(batch, seq, d_model)
    segf = segment_ids.astype(jnp.float32).reshape(batch, seq, 1)
    return _tc_addnorm(gat, segf, pos_emb, seg_emb, gamma, beta)


# speedup vs baseline: 4.1630x; 1.0186x over previous
import jax
import jax.numpy as jnp
from jax import lax
from jax.experimental import pallas as pl
from jax.experimental.pallas import tpu as pltpu
from jax.experimental.pallas import tpu_sc as plsc

NC = 2
NS = 16
NW = NC * NS

CHG = 64
RPB = 2
K = 4
LN_EPS = 1e-12


def _sc_gather(ids, tok_emb):
    tokens, = ids.shape
    vocab, d_model = tok_emb.shape
    assert tokens % NW == 0
    tpw = tokens // NW
    assert tpw % CHG == 0
    nch = tpw // CHG

    mesh = plsc.VectorSubcoreMesh(
        core_axis_name="c", subcore_axis_name="s", num_cores=NC, num_subcores=NS
    )

    def body(ids_h, tok_h, out_h, idx_all, rows_v, gsem, osem):
        w = lax.axis_index("s") * NC + lax.axis_index("c")
        wbase = w * tpw
        pltpu.sync_copy(ids_h.at[pl.ds(wbase, tpw)], idx_all)
        pend_g = None
        pend_o = [None, None]
        for c in range(nch + 1):
            p = c & 1
            if pend_o[p] is not None:
                pend_o[p].wait()
                pend_o[p] = None
            g = None
            if c < nch:
                g = pltpu.async_copy(
                    tok_h.at[idx_all.at[pl.ds(c * CHG, CHG)]], rows_v.at[p], gsem
                )
            if pend_g is not None:
                pend_g.wait()
                pend_o[1 - p] = pltpu.async_copy(
                    rows_v.at[1 - p],
                    out_h.at[pl.ds(wbase + (c - 1) * CHG, CHG)],
                    osem,
                )
            pend_g = g
        for p in range(2):
            if pend_o[p] is not None:
                pend_o[p].wait()

    return pl.kernel(
        body,
        out_type=jax.ShapeDtypeStruct((tokens, d_model), jnp.float32),
        mesh=mesh,
        compiler_params=pltpu.CompilerParams(needs_layout_passes=False),
        scratch_types=[
            pltpu.VMEM((tpw,), jnp.int32),
            pltpu.VMEM((2, CHG, d_model), jnp.float32),
            pltpu.SemaphoreType.DMA,
            pltpu.SemaphoreType.DMA,
        ],
    )(ids, tok_emb)


def _tc_addnorm(prev_out, gat_k, segf_k, pos_emb, seg_emb, gamma, beta,
                batch, base_rows):
    rows_k, seq, d_model = gat_k.shape

    def body(*refs):
        gat_ref, segf_ref, pos_ref, sege_ref, gam_ref, bet_ref = refs[-7:-1]
        out_ref = refs[-1]
        base = pos_ref[...] + sege_ref[0, :][None, :]
        diff = (sege_ref[1, :] - sege_ref[0, :])[None, None, :]
        emb = gat_ref[...] + base[None, :, :] + segf_ref[...] * diff
        mean = jnp.mean(emb, axis=-1, keepdims=True)
        cent = emb - mean
        var = jnp.mean(cent * cent, axis=-1, keepdims=True)
        rstd = lax.rsqrt(var + LN_EPS)
        out_ref[...] = (cent * rstd) * gam_ref[0, :][None, None, :] \
            + bet_ref[0, :][None, None, :]

    base_blk = base_rows // RPB
    in_specs = [
        pl.BlockSpec((RPB, seq, d_model), lambda i: (i, 0, 0)),
        pl.BlockSpec((RPB, seq, 1), lambda i: (i, 0, 0)),
        pl.BlockSpec((seq, d_model), lambda i: (0, 0)),
        pl.BlockSpec((2, d_model), lambda i: (0, 0)),
        pl.BlockSpec((1, d_model), lambda i: (0, 0)),
        pl.BlockSpec((1, d_model), lambda i: (0, 0)),
    ]
    args = (gat_k, segf_k, pos_emb, seg_emb,
            gamma.reshape(1, -1), beta.reshape(1, -1))
    aliases = {}
    if prev_out is not None:
        in_specs = [pl.BlockSpec(memory_space=pl.ANY)] + in_specs
        args = (prev_out,) + args
        aliases = {0: 0}

    return pl.pallas_call(
        body if prev_out is not None else (lambda *r: body(None, *r)),
        grid=(rows_k // RPB,),
        in_specs=in_specs,
        out_specs=pl.BlockSpec((RPB, seq, d_model),
                               lambda i: (i + base_blk, 0, 0)),
        out_shape=jax.ShapeDtypeStruct((batch, seq, d_model), jnp.float32),
        input_output_aliases=aliases,
    )(*args)


def kernel(input_ids, segment_ids, tok_emb, pos_emb, seg_emb, gamma, beta):
    batch, seq = input_ids.shape
    _, d_model = tok_emb.shape
    assert batch % (K * RPB) == 0
    bk = batch // K
    ids = input_ids.reshape(-1).astype(jnp.int32)
    segf = segment_ids.astype(jnp.float32).reshape(batch, seq, 1)

    gats = [
        _sc_gather(ids[k * bk * seq:(k + 1) * bk * seq], tok_emb)
        .reshape(bk, seq, d_model)
        for k in range(K)
    ]
    out = None
    for k in range(K):
        out = _tc_addnorm(out, gats[k], segf[k * bk:(k + 1) * bk],
                          pos_emb, seg_emb, gamma, beta, batch, k * bk)
    return out


# speedup vs baseline: 4.1722x; 1.0022x over previous
import jax
import jax.numpy as jnp
from jax import lax
from jax.experimental import pallas as pl
from jax.experimental.pallas import tpu as pltpu
from jax.experimental.pallas import tpu_sc as plsc

NC = 2
NS = 16
NW = NC * NS

CHG = 64
RPB = 2
K = 4
LN_EPS = 1e-12


def _sc_gather(ids, tok_emb):
    tokens, = ids.shape
    vocab, d_model = tok_emb.shape
    assert tokens % NW == 0
    tpw = tokens // NW
    assert tpw % CHG == 0
    nch = tpw // CHG

    mesh = plsc.VectorSubcoreMesh(
        core_axis_name="c", subcore_axis_name="s", num_cores=NC, num_subcores=NS
    )

    def body(ids_h, tok_h, out_h, idx_all, rows_v, gsem, osem):
        w = lax.axis_index("s") * NC + lax.axis_index("c")
        wbase = w * tpw
        pltpu.sync_copy(ids_h.at[pl.ds(wbase, tpw)], idx_all)
        pend_g = None
        pend_o = [None, None]
        for c in range(nch + 1):
            p = c & 1
            if pend_o[p] is not None:
                pend_o[p].wait()
                pend_o[p] = None
            g = None
            if c < nch:
                g = pltpu.async_copy(
                    tok_h.at[idx_all.at[pl.ds(c * CHG, CHG)]], rows_v.at[p], gsem
                )
            if pend_g is not None:
                pend_g.wait()
                pend_o[1 - p] = pltpu.async_copy(
                    rows_v.at[1 - p],
                    out_h.at[pl.ds(wbase + (c - 1) * CHG, CHG)],
                    osem,
                )
            pend_g = g
        for p in range(2):
            if pend_o[p] is not None:
                pend_o[p].wait()

    return pl.kernel(
        body,
        out_type=jax.ShapeDtypeStruct((tokens, d_model), jnp.float32),
        mesh=mesh,
        compiler_params=pltpu.CompilerParams(needs_layout_passes=False),
        scratch_types=[
            pltpu.VMEM((tpw,), jnp.int32),
            pltpu.VMEM((2, CHG, d_model), jnp.float32),
            pltpu.SemaphoreType.DMA,
            pltpu.SemaphoreType.DMA,
        ],
    )(ids, tok_emb)


def _tc_addnorm(prev_out, gat_k, segf, pos_emb, seg_emb, gamma, beta,
                batch, base_rows):
    rows_k, seq, d_model = gat_k.shape

    def body(*refs):
        gat_ref, segf_ref, pos_ref, sege_ref, gam_ref, bet_ref = refs[-7:-1]
        out_ref = refs[-1]
        base = pos_ref[...] + sege_ref[0, :][None, :]
        diff = (sege_ref[1, :] - sege_ref[0, :])[None, None, :]
        emb = gat_ref[...] + base[None, :, :] + segf_ref[...] * diff
        mean = jnp.mean(emb, axis=-1, keepdims=True)
        cent = emb - mean
        var = jnp.mean(cent * cent, axis=-1, keepdims=True)
        rstd = lax.rsqrt(var + LN_EPS)
        out_ref[...] = (cent * rstd) * gam_ref[0, :][None, None, :] \
            + bet_ref[0, :][None, None, :]

    base_blk = base_rows // RPB
    in_specs = [
        pl.BlockSpec((RPB, seq, d_model), lambda i: (i, 0, 0)),
        pl.BlockSpec((RPB, seq, 1), lambda i: (i + base_blk, 0, 0)),
        pl.BlockSpec((seq, d_model), lambda i: (0, 0)),
        pl.BlockSpec((2, d_model), lambda i: (0, 0)),
        pl.BlockSpec((1, d_model), lambda i: (0, 0)),
        pl.BlockSpec((1, d_model), lambda i: (0, 0)),
    ]
    args = (gat_k, segf, pos_emb, seg_emb,
            gamma.reshape(1, -1), beta.reshape(1, -1))
    aliases = {}
    if prev_out is not None:
        in_specs = [pl.BlockSpec(memory_space=pl.ANY)] + in_specs
        args = (prev_out,) + args
        aliases = {0: 0}

    return pl.pallas_call(
        body,
        grid=(rows_k // RPB,),
        in_specs=in_specs,
        out_specs=pl.BlockSpec((RPB, seq, d_model),
                               lambda i: (i + base_blk, 0, 0)),
        out_shape=jax.ShapeDtypeStruct((batch, seq, d_model), jnp.float32),
        input_output_aliases=aliases,
    )(*args)


def kernel(input_ids, segment_ids, tok_emb, pos_emb, seg_emb, gamma, beta):
    batch, seq = input_ids.shape
    _, d_model = tok_emb.shape
    assert batch % (K * RPB) == 0
    bk = batch // K
    ids = input_ids.reshape(-1).astype(jnp.int32)
    segf = segment_ids.astype(jnp.float32).reshape(batch, seq, 1)

    gats = [
        _sc_gather(ids[k * bk * seq:(k + 1) * bk * seq], tok_emb)
        .reshape(bk, seq, d_model)
        for k in range(K)
    ]
    out = None
    for k in range(K):
        out = _tc_addnorm(out, gats[k], segf,
                          pos_emb, seg_emb, gamma, beta, batch, k * bk)
    return out
